# baseline (speedup 1.0000x reference)
import functools

import jax
import jax.numpy as jnp
from jax import lax
from jax.experimental import pallas as pl
from jax.experimental.pallas import tpu as pltpu

H = 128
W = 128
TILE = 32
NT = 4
PIX = TILE * TILE
CG = 128

_LOG2E = 1.4426950408889634
_LN2 = 0.6931471805599453


def _fexp(x):
    x = jnp.maximum(x, -87.0)
    y = x * _LOG2E
    z = y + 0.5
    t = z.astype(jnp.int32)
    k = t - (t.astype(jnp.float32) > z).astype(jnp.int32)
    u = (y - k.astype(jnp.float32)) * _LN2
    p = 1.0 + u * (1.0 + u * (0.5 + u * (
        1.0 / 6.0 + u * (1.0 / 24.0 + u * (1.0 / 120.0 + u * (1.0 / 720.0))))))
    return lax.bitcast_convert_type(
        lax.bitcast_convert_type(p, jnp.int32) + (k << 23), jnp.float32)


def _raster_kernel(n, nneg, rows_ref, cols_ref, neg_ref, out_ref, nacc_ref):
    i = pl.program_id(0)
    ty = i // NT
    tx = i % NT
    w0 = (tx * TILE).astype(jnp.float32)
    h0 = (ty * TILE).astype(jnp.float32)

    pi = lax.broadcasted_iota(jnp.int32, (1, PIX), 1)
    xf = (pi % TILE).astype(jnp.float32) + w0
    yf = (pi // TILE).astype(jnp.float32) + h0

    mx = rows_ref[0:1, :]
    my = rows_ref[1:2, :]
    c00 = rows_ref[2:3, :]
    c01 = rows_ref[3:4, :]
    c11 = rows_ref[4:5, :]
    det = c00 * c11 - c01 * c01
    mid = 0.5 * (c00 + c11)
    s = jnp.sqrt(jnp.clip(mid * mid - det, 0.1, None))
    r = 3.0 * jnp.ceil(jnp.sqrt(mid + s))
    tl0 = jnp.maximum(jnp.clip(mx - r, 0.0, W - 1.0), w0)
    br0 = jnp.minimum(jnp.clip(mx + r, 0.0, W - 1.0), w0 + (TILE - 1.0))
    tl1 = jnp.maximum(jnp.clip(my - r, 0.0, H - 1.0), h0)
    br1 = jnp.minimum(jnp.clip(my + r, 0.0, H - 1.0), h0 + (TILE - 1.0))
    pmask = ((br0 > tl0) & (br1 > tl1)).astype(jnp.float32)

    li = lax.broadcasted_iota(jnp.int32, (1, n), 1)
    x = pmask
    sh = 1
    while sh < n:
        x = x + jnp.where(li >= sh, pltpu.roll(x, sh, 1), 0.0)
        sh *= 2
    rank = x - pmask
    cnt = x[0, n - 1].astype(jnp.int32)
    nchunks = (cnt + (CG - 1)) // CG

    nmx = neg_ref[0:1, :]
    nmy = neg_ref[1:2, :]
    n00 = neg_ref[2:3, :]
    n01 = neg_ref[3:4, :]
    n02 = neg_ref[4:5, :]
    n11 = neg_ref[5:6, :]
    n12 = neg_ref[6:7, :]
    n22 = neg_ref[7:8, :]
    nop = neg_ref[8:9, :]
    nd = neg_ref[9:10, :]

    a00 = n11 * n22 - n12 * n12
    a01 = -(n01 * n22 - n02 * n12)
    a02 = n01 * n12 - n02 * n11
    a11 = n00 * n22 - n02 * n02
    a12 = -(n00 * n12 - n01 * n02)
    a22 = n00 * n11 - n01 * n01
    ndet = n00 * a00 + n01 * a01 + n02 * a02
    i00 = a00 / ndet
    i01 = a01 / ndet
    i02 = a02 / ndet
    i11 = a11 / ndet
    i12 = a12 / ndet
    i22 = a22 / ndet

    ndetr = n00 * n11 - n01 * n01
    nmid = 0.5 * (n00 + n11)
    ns = jnp.sqrt(jnp.clip(nmid * nmid - ndetr, 0.1, None))
    nr = 3.0 * jnp.ceil(jnp.sqrt(nmid + ns))
    ntl0 = jnp.maximum(jnp.clip(nmx - nr, 0.0, W - 1.0), w0)
    nbr0 = jnp.minimum(jnp.clip(nmx + nr, 0.0, W - 1.0), w0 + (TILE - 1.0))
    ntl1 = jnp.maximum(jnp.clip(nmy - nr, 0.0, H - 1.0), h0)
    nbr1 = jnp.minimum(jnp.clip(nmy + nr, 0.0, H - 1.0), h0 + (TILE - 1.0))
    nmask = ((nbr0 > ntl0) & (nbr1 > ntl1)).astype(jnp.float32)
    nopm = nop * nmask

    oi = lax.broadcasted_iota(jnp.int32, (CG, 1), 0).astype(jnp.float32)
    si = lax.broadcasted_iota(jnp.int32, (CG, PIX), 0)

    def chunk_body(c, carry):
        Tt, acc, o0, o1, o2, od = carry
        basef = (c * CG).astype(jnp.float32)
        G = jnp.where((rank - basef == oi) & (pmask > 0.0), 1.0, 0.0)
        gat = lax.dot_general(
            G, cols_ref[...], (((1,), (0,)), ((), ())),
            preferred_element_type=jnp.float32,
            precision=lax.Precision.HIGHEST)
        mxg = gat[:, 0:1]
        myg = gat[:, 1:2]
        g00 = gat[:, 2:3]
        g01 = gat[:, 3:4]
        g11 = gat[:, 4:5]
        opg = gat[:, 5:6]
        cl0 = gat[:, 6:7]
        cl1 = gat[:, 7:8]
        cl2 = gat[:, 8:9]
        dg = gat[:, 9:10]

        gdet = g00 * g11 - g01 * g01
        gdet = jnp.where(gdet == 0.0, 1.0, gdet)
        k00 = g11 / gdet
        k11 = g00 / gdet
        k01 = -g01 / gdet

        dx0 = xf - mxg
        dx1 = yf - myg
        ge = _fexp(-0.5 * (dx0 * dx0 * k00 + dx1 * dx1 * k11)
                   - k01 * dx0 * dx1)
        a = jnp.minimum(ge * opg, 0.99)

        nacc_ref[...] = jnp.zeros((CG, PIX), jnp.float32)
        for k in range(nneg):
            @pl.when(nopm[0, k] > 0.0)
            def _(k=k):
                nx0 = xf - nmx[0, k]
                nx1 = yf - nmy[0, k]
                Pk = -0.5 * (nx0 * nx0 * i00[0, k] + nx1 * nx1 * i11[0, k]
                             + 2.0 * i01[0, k] * nx0 * nx1)
                Lk = -(nx0 * i02[0, k] + nx1 * i12[0, k])
                d2 = dg - nd[0, k]
                e = Pk + Lk * d2 + (-0.5 * i22[0, k]) * (d2 * d2)
                nacc_ref[...] += nopm[0, k] * _fexp(e)
        a = jnp.minimum(a * jnp.clip(1.0 - nacc_ref[...], 0.0, 1.0), 0.99)

        q = 1.0 - a
        e = jnp.where(si >= 1, pltpu.roll(q, 1, 0), 1.0)
        sh = 1
        while sh < CG:
            e = e * jnp.where(si >= sh, pltpu.roll(e, sh, 0), 1.0)
            sh *= 2

        wgt = Tt * e * a
        acc = acc + jnp.sum(wgt, axis=0, keepdims=True)
        o0 = o0 + jnp.sum(wgt * cl0, axis=0, keepdims=True)
        o1 = o1 + jnp.sum(wgt * cl1, axis=0, keepdims=True)
        o2 = o2 + jnp.sum(wgt * cl2, axis=0, keepdims=True)
        od = od + jnp.sum(wgt * dg, axis=0, keepdims=True)
        Tt = Tt * (e[CG - 1:CG, :] * q[CG - 1:CG, :])
        return Tt, acc, o0, o1, o2, od

    init = (jnp.ones((1, PIX), jnp.float32),) + tuple(
        jnp.zeros((1, PIX), jnp.float32) for _ in range(5))
    Tt, acc, o0, o1, o2, od = lax.fori_loop(0, nchunks, chunk_body, init)

    bg = 1.0 - acc
    out = jnp.concatenate(
        [o0 + bg, o1 + bg, o2 + bg, od, acc, jnp.zeros((3, PIX), jnp.float32)],
        axis=0)
    out_ref[0] = out


def kernel(means2D, cov2d, color, opacity, depths, neg_means2D, neg_cov3d,
           neg_opacity, neg_depths):
    n = means2D.shape[0]
    nneg = neg_means2D.shape[0]

    order = jnp.argsort(depths)
    sm = jnp.take(means2D, order, axis=0)
    scov = jnp.take(cov2d, order, axis=0)
    scol = jnp.take(color, order, axis=0)
    sop = jnp.take(opacity, order, axis=0)
    sd = jnp.take(depths, order, axis=0)

    rows = jnp.concatenate([
        jnp.stack([sm[:, 0], sm[:, 1],
                   scov[:, 0, 0], scov[:, 0, 1], scov[:, 1, 1],
                   sop[:, 0],
                   scol[:, 0], scol[:, 1], scol[:, 2], sd], axis=0),
        jnp.zeros((6, n), jnp.float32)], axis=0)
    cols = rows.T
    neg = jnp.concatenate([
        jnp.stack([neg_means2D[:, 0], neg_means2D[:, 1],
                   neg_cov3d[:, 0, 0], neg_cov3d[:, 0, 1], neg_cov3d[:, 0, 2],
                   neg_cov3d[:, 1, 1], neg_cov3d[:, 1, 2], neg_cov3d[:, 2, 2],
                   neg_opacity[:, 0], neg_depths], axis=0),
        jnp.zeros((6, nneg), jnp.float32)], axis=0)

    body = functools.partial(_raster_kernel, n, nneg)
    res = pl.pallas_call(
        body,
        grid=(NT * NT,),
        in_specs=[
            pl.BlockSpec((16, n), lambda i: (0, 0)),
            pl.BlockSpec((n, 16), lambda i: (0, 0)),
            pl.BlockSpec((16, nneg), lambda i: (0, 0)),
        ],
        out_specs=pl.BlockSpec((1, 8, PIX), lambda i: (i, 0, 0)),
        out_shape=jax.ShapeDtypeStruct((NT * NT, 8, PIX), jnp.float32),
        scratch_shapes=[pltpu.VMEM((CG, PIX), jnp.float32)],
        compiler_params=pltpu.CompilerParams(
            dimension_semantics=("parallel",)),
    )(rows, cols, neg)

    return (res[:, :5, :]
            .reshape(NT, NT, 5, TILE, TILE)
            .transpose(0, 3, 1, 4, 2)
            .reshape(H, W, 5))


# speedup vs baseline: 1.1055x; 1.1055x over previous
import functools

import jax
import jax.numpy as jnp
from jax import lax
from jax.experimental import pallas as pl
from jax.experimental.pallas import tpu as pltpu

H = 128
W = 128
TILE = 32
NT = 4
PIX = TILE * TILE
CG = 128


def _raster_kernel(n, nneg, rows_ref, cols_ref, neg_ref, out_ref, nacc_ref):
    i = pl.program_id(0)
    ty = i // NT
    tx = i % NT
    w0 = (tx * TILE).astype(jnp.float32)
    h0 = (ty * TILE).astype(jnp.float32)

    pi = lax.broadcasted_iota(jnp.int32, (1, PIX), 1)
    xf = (pi % TILE).astype(jnp.float32) + w0
    yf = (pi // TILE).astype(jnp.float32) + h0

    mx = rows_ref[0:1, :]
    my = rows_ref[1:2, :]
    c00 = rows_ref[2:3, :]
    c01 = rows_ref[3:4, :]
    c11 = rows_ref[4:5, :]
    det = c00 * c11 - c01 * c01
    mid = 0.5 * (c00 + c11)
    s = jnp.sqrt(jnp.clip(mid * mid - det, 0.1, None))
    r = 3.0 * jnp.ceil(jnp.sqrt(mid + s))
    tl0 = jnp.maximum(jnp.clip(mx - r, 0.0, W - 1.0), w0)
    br0 = jnp.minimum(jnp.clip(mx + r, 0.0, W - 1.0), w0 + (TILE - 1.0))
    tl1 = jnp.maximum(jnp.clip(my - r, 0.0, H - 1.0), h0)
    br1 = jnp.minimum(jnp.clip(my + r, 0.0, H - 1.0), h0 + (TILE - 1.0))
    pmask = ((br0 > tl0) & (br1 > tl1)).astype(jnp.float32)

    li = lax.broadcasted_iota(jnp.int32, (1, n), 1)
    x = pmask
    sh = 1
    while sh < n:
        x = x + jnp.where(li >= sh, pltpu.roll(x, sh, 1), 0.0)
        sh *= 2
    rank = x - pmask
    cnt = x[0, n - 1].astype(jnp.int32)
    nchunks = (cnt + (CG - 1)) // CG

    nmx = neg_ref[0:1, :]
    nmy = neg_ref[1:2, :]
    n00 = neg_ref[2:3, :]
    n01 = neg_ref[3:4, :]
    n02 = neg_ref[4:5, :]
    n11 = neg_ref[5:6, :]
    n12 = neg_ref[6:7, :]
    n22 = neg_ref[7:8, :]
    nop = neg_ref[8:9, :]
    nd = neg_ref[9:10, :]

    a00 = n11 * n22 - n12 * n12
    a01 = -(n01 * n22 - n02 * n12)
    a02 = n01 * n12 - n02 * n11
    a11 = n00 * n22 - n02 * n02
    a12 = -(n00 * n12 - n01 * n02)
    a22 = n00 * n11 - n01 * n01
    ndet = n00 * a00 + n01 * a01 + n02 * a02
    i00 = a00 / ndet
    i01 = a01 / ndet
    i02 = a02 / ndet
    i11 = a11 / ndet
    i12 = a12 / ndet
    i22 = a22 / ndet

    ndetr = n00 * n11 - n01 * n01
    nmid = 0.5 * (n00 + n11)
    ns = jnp.sqrt(jnp.clip(nmid * nmid - ndetr, 0.1, None))
    nr = 3.0 * jnp.ceil(jnp.sqrt(nmid + ns))
    ntl0 = jnp.maximum(jnp.clip(nmx - nr, 0.0, W - 1.0), w0)
    nbr0 = jnp.minimum(jnp.clip(nmx + nr, 0.0, W - 1.0), w0 + (TILE - 1.0))
    ntl1 = jnp.maximum(jnp.clip(nmy - nr, 0.0, H - 1.0), h0)
    nbr1 = jnp.minimum(jnp.clip(nmy + nr, 0.0, H - 1.0), h0 + (TILE - 1.0))
    nmask = ((nbr0 > ntl0) & (nbr1 > ntl1)).astype(jnp.float32)
    nopm = nop * nmask

    oi = lax.broadcasted_iota(jnp.int32, (CG, 1), 0).astype(jnp.float32)
    si = lax.broadcasted_iota(jnp.int32, (CG, PIX), 0)

    def chunk_body(c, carry):
        Tt, acc, o0, o1, o2, od = carry
        basef = (c * CG).astype(jnp.float32)
        G = jnp.where((rank - basef == oi) & (pmask > 0.0), 1.0, 0.0)
        gat = lax.dot_general(
            G, cols_ref[...], (((1,), (0,)), ((), ())),
            preferred_element_type=jnp.float32,
            precision=lax.Precision.HIGHEST)
        mxg = gat[:, 0:1]
        myg = gat[:, 1:2]
        g00 = gat[:, 2:3]
        g01 = gat[:, 3:4]
        g11 = gat[:, 4:5]
        opg = gat[:, 5:6]
        cl0 = gat[:, 6:7]
        cl1 = gat[:, 7:8]
        cl2 = gat[:, 8:9]
        dg = gat[:, 9:10]

        gdet = g00 * g11 - g01 * g01
        gdet = jnp.where(gdet == 0.0, 1.0, gdet)
        k00 = g11 / gdet
        k11 = g00 / gdet
        k01 = -g01 / gdet

        dx0 = xf - mxg
        dx1 = yf - myg
        ge = jnp.exp(-0.5 * (dx0 * dx0 * k00 + dx1 * dx1 * k11)
                     - k01 * dx0 * dx1)
        a = jnp.minimum(ge * opg, 0.99)

        nacc_ref[...] = jnp.zeros((CG, PIX), jnp.float32)
        for k in range(nneg):
            @pl.when(nopm[0, k] > 0.0)
            def _(k=k):
                nx0 = xf - nmx[0, k]
                nx1 = yf - nmy[0, k]
                Pk = -0.5 * (nx0 * nx0 * i00[0, k] + nx1 * nx1 * i11[0, k]
                             + 2.0 * i01[0, k] * nx0 * nx1)
                Lk = -(nx0 * i02[0, k] + nx1 * i12[0, k])
                d2 = dg - nd[0, k]
                e = Pk + Lk * d2 + (-0.5 * i22[0, k]) * (d2 * d2)
                nacc_ref[...] += nopm[0, k] * jnp.exp(e)
        a = jnp.minimum(a * jnp.clip(1.0 - nacc_ref[...], 0.0, 1.0), 0.99)

        q = 1.0 - a
        e = jnp.where(si >= 1, pltpu.roll(q, 1, 0), 1.0)
        sh = 1
        while sh < CG:
            e = e * jnp.where(si >= sh, pltpu.roll(e, sh, 0), 1.0)
            sh *= 2

        wgt = Tt * e * a
        acc = acc + jnp.sum(wgt, axis=0, keepdims=True)
        o0 = o0 + jnp.sum(wgt * cl0, axis=0, keepdims=True)
        o1 = o1 + jnp.sum(wgt * cl1, axis=0, keepdims=True)
        o2 = o2 + jnp.sum(wgt * cl2, axis=0, keepdims=True)
        od = od + jnp.sum(wgt * dg, axis=0, keepdims=True)
        Tt = Tt * (e[CG - 1:CG, :] * q[CG - 1:CG, :])
        return Tt, acc, o0, o1, o2, od

    init = (jnp.ones((1, PIX), jnp.float32),) + tuple(
        jnp.zeros((1, PIX), jnp.float32) for _ in range(5))
    Tt, acc, o0, o1, o2, od = lax.fori_loop(0, nchunks, chunk_body, init)

    bg = 1.0 - acc
    out = jnp.concatenate(
        [o0 + bg, o1 + bg, o2 + bg, od, acc, jnp.zeros((3, PIX), jnp.float32)],
        axis=0)
    out_ref[0] = out


def kernel(means2D, cov2d, color, opacity, depths, neg_means2D, neg_cov3d,
           neg_opacity, neg_depths):
    n = means2D.shape[0]
    nneg = neg_means2D.shape[0]

    order = jnp.argsort(depths)
    sm = jnp.take(means2D, order, axis=0)
    scov = jnp.take(cov2d, order, axis=0)
    scol = jnp.take(color, order, axis=0)
    sop = jnp.take(opacity, order, axis=0)
    sd = jnp.take(depths, order, axis=0)

    rows = jnp.concatenate([
        jnp.stack([sm[:, 0], sm[:, 1],
                   scov[:, 0, 0], scov[:, 0, 1], scov[:, 1, 1],
                   sop[:, 0],
                   scol[:, 0], scol[:, 1], scol[:, 2], sd], axis=0),
        jnp.zeros((6, n), jnp.float32)], axis=0)
    cols = rows.T
    neg = jnp.concatenate([
        jnp.stack([neg_means2D[:, 0], neg_means2D[:, 1],
                   neg_cov3d[:, 0, 0], neg_cov3d[:, 0, 1], neg_cov3d[:, 0, 2],
                   neg_cov3d[:, 1, 1], neg_cov3d[:, 1, 2], neg_cov3d[:, 2, 2],
                   neg_opacity[:, 0], neg_depths], axis=0),
        jnp.zeros((6, nneg), jnp.float32)], axis=0)

    body = functools.partial(_raster_kernel, n, nneg)
    res = pl.pallas_call(
        body,
        grid=(NT * NT,),
        in_specs=[
            pl.BlockSpec((16, n), lambda i: (0, 0)),
            pl.BlockSpec((n, 16), lambda i: (0, 0)),
            pl.BlockSpec((16, nneg), lambda i: (0, 0)),
        ],
        out_specs=pl.BlockSpec((1, 8, PIX), lambda i: (i, 0, 0)),
        out_shape=jax.ShapeDtypeStruct((NT * NT, 8, PIX), jnp.float32),
        scratch_shapes=[pltpu.VMEM((CG, PIX), jnp.float32)],
        compiler_params=pltpu.CompilerParams(
            dimension_semantics=("parallel",)),
    )(rows, cols, neg)

    return (res[:, :5, :]
            .reshape(NT, NT, 5, TILE, TILE)
            .transpose(0, 3, 1, 4, 2)
            .reshape(H, W, 5))
